# baseline (speedup 1.0000x reference)
import functools

import jax
import jax.numpy as jnp
from jax import lax
from jax.experimental import pallas as pl
from jax.experimental.pallas import tpu as pltpu
from jax.experimental.pallas import tpu_sc as plsc

_F = 128
_RES = 512
_N = 65536
_NC, _NS, _L = 2, 16, 16
_NW = _NC * _NS
_ROWS_W = _N // _NW
_CROWS = 64
_NCHUNK = _ROWS_W // _CROWS
_VPR = _F // _L

_MESH = plsc.VectorSubcoreMesh(core_axis_name="c", subcore_axis_name="s")

_f32 = jnp.float32
_i32 = jnp.int32


@functools.partial(
    pl.kernel,
    out_type=jax.ShapeDtypeStruct((_N, _F), _f32),
    mesh=_MESH,
    scratch_types=[
        pltpu.VMEM((_CROWS, _F), _f32),
        pltpu.VMEM((_CROWS, _F), _f32),
        pltpu.VMEM((_CROWS, _F), _i32),
        pltpu.VMEM((_CROWS, _F), _i32),
        pltpu.VMEM((_CROWS, _F), _i32),
        pltpu.VMEM((_CROWS, _F), _i32),
        pltpu.VMEM((_CROWS, _F), _f32),
        pltpu.VMEM((_CROWS, _F), _f32),
        pltpu.VMEM((_CROWS, _F), _f32),
        pltpu.VMEM((_CROWS, _F), _f32),
        pltpu.VMEM((_CROWS, _F), _f32),
        pltpu.VMEM((_CROWS, _F), _f32),
        pltpu.VMEM((_CROWS, _F), _f32),
        pltpu.SemaphoreType.DMA,
    ],
)
def _sc_sample(x_hbm, y_hbm, img_hbm, out_hbm,
               xv, yv, idx00, idx01, idx10, idx11,
               v00, v01, v10, v11, xwv, ywv, outv, sem):
    wid = lax.axis_index("s") * _NC + lax.axis_index("c")
    row0 = wid * _ROWS_W

    lane = lax.iota(_i32, _L)

    def chunk_body(k, _):
        rbase = row0 + k * _CROWS
        pltpu.sync_copy(x_hbm.at[pl.ds(rbase, _CROWS)], xv)
        pltpu.sync_copy(y_hbm.at[pl.ds(rbase, _CROWS)], yv)

        def idx_row(j, _):
            for c in range(_VPR):
                sl = pl.ds(c * _L, _L)
                x = xv[j, sl] - 0.5
                y = yv[j, sl] - 0.5
                xi = x.astype(_i32)
                yi = y.astype(_i32)
                x0 = jnp.where(xi.astype(_f32) > x, xi - 1, xi)
                y0 = jnp.where(yi.astype(_f32) > y, yi - 1, yi)
                xwv[j, sl] = x - x0.astype(_f32)
                ywv[j, sl] = y - y0.astype(_f32)
                x0m = jnp.bitwise_and(x0, _RES - 1)
                x1m = jnp.bitwise_and(x0 + 1, _RES - 1)
                y0s = jnp.bitwise_and(y0, _RES - 1) * _RES
                y1s = jnp.bitwise_and(y0 + 1, _RES - 1) * _RES
                fbase = (lane + c * _L) * (_RES * _RES)
                t0 = fbase + y0s
                t1 = fbase + y1s
                idx00[j, sl] = t0 + x0m
                idx01[j, sl] = t0 + x1m
                idx10[j, sl] = t1 + x0m
                idx11[j, sl] = t1 + x1m
            return 0

        lax.fori_loop(0, _CROWS, idx_row, 0)

        def gather_row(j, _):
            pltpu.async_copy(img_hbm.at[idx00.at[j]], v00.at[j], sem)
            pltpu.async_copy(img_hbm.at[idx01.at[j]], v01.at[j], sem)
            pltpu.async_copy(img_hbm.at[idx10.at[j]], v10.at[j], sem)
            pltpu.async_copy(img_hbm.at[idx11.at[j]], v11.at[j], sem)
            return 0

        lax.fori_loop(0, _CROWS, gather_row, 0)

        for vbuf in (v00, v01, v10, v11):
            pltpu.make_async_copy(out_hbm.at[pl.ds(0, _CROWS)], vbuf, sem).wait()

        def lerp_row(j, _):
            for c in range(_VPR):
                sl = pl.ds(c * _L, _L)
                a00 = v00[j, sl]
                a01 = v01[j, sl]
                a10 = v10[j, sl]
                a11 = v11[j, sl]
                xw = xwv[j, sl]
                yw = ywv[j, sl]
                i0 = a00 + (a01 - a00) * xw
                i1 = a10 + (a11 - a10) * xw
                outv[j, sl] = i0 + (i1 - i0) * yw
            return 0

        lax.fori_loop(0, _CROWS, lerp_row, 0)

        pltpu.sync_copy(outv, out_hbm.at[pl.ds(rbase, _CROWS)])
        return 0

    lax.fori_loop(0, _NCHUNK, chunk_body, 0)


def kernel(coords, image):
    x = coords[:, :, 0]
    y = coords[:, :, 1]
    img = image.reshape(-1)
    return _sc_sample(x, y, img)
